# baseline (speedup 1.0000x reference)
import jax
import jax.numpy as jnp
from jax import lax
from jax.experimental import pallas as pl
from jax.experimental.pallas import tpu as pltpu
from jax.experimental.pallas import tpu_sc as plsc

B, J, O = 64, 50, 50
D = 128
JO = J * O
N = B * JO
NC, NS = 2, 16
NW = NC * NS
BPW = B // NW
CH = 200
NCB = 13
JOPAD = 2512
WIDX = BPW * JOPAD
TMR = 200


def _prep_body(job_ref, mach_ref, seq_ref, wt_ref, bt_ref, wp_ref, bp_ref,
               base_ref, tm_ref):
    f32 = jnp.float32
    wp = wp_ref[...]
    w1 = wp[0:128, :]
    w2 = wp[128:256, :]
    w3 = wp[256:384, :]
    w4 = wp[384:512, :]
    const = jnp.dot(bt_ref[...], w4, preferred_element_type=f32) + bp_ref[...]
    a = jnp.dot(job_ref[...][:J], w1, preferred_element_type=f32) + const
    c = jnp.dot(seq_ref[...][:O], w3, preferred_element_type=f32)
    base_ref[...] = (a[:, None, :] + c[None, :, :]).reshape(JO, D)
    m2 = jnp.dot(mach_ref[...], w2, preferred_element_type=f32)
    v = jnp.dot(wt_ref[...], w4, preferred_element_type=f32)
    t_col = lax.broadcasted_iota(jnp.int32, (100, 1), 0).astype(f32)
    tm_ref[...] = jnp.concatenate([t_col * v.reshape(1, D), m2], axis=0)


def _sc_body(idx_hbm, base_hbm, tm_hbm, out_hbm,
             idx_v, buf0, buf1, buf2, sptm, spb,
             sb0, sb1, sb2, sg0, sg1, sg2, so0, so1, so2):
    cid = lax.axis_index("c")
    sid = lax.axis_index("s")
    wid = sid * NC + cid
    b0 = wid * BPW

    pltpu.sync_copy(idx_hbm.at[pl.ds(b0 * JOPAD, WIDX)],
                    idx_v.at[pl.ds(0, WIDX)])
    pltpu.sync_copy(idx_hbm.at[pl.ds(B * JOPAD + b0 * JOPAD, WIDX)],
                    idx_v.at[pl.ds(WIDX, WIDX)])

    @pl.when(sid == 0)
    def _stage():
        pltpu.sync_copy(tm_hbm, sptm)
        pltpu.sync_copy(base_hbm, spb)
    plsc.subcore_barrier()

    bufs = (buf0, buf1, buf2)
    semb = (sb0, sb1, sb2)
    semg = (sg0, sg1, sg2)
    semo = (so0, so1, so2)

    chunks = [(b2, c * CH, CH if c < NCB - 1 else JO - (NCB - 1) * CH)
              for b2 in range(BPW) for c in range(NCB)]
    ncv = len(chunks)

    baseh = [None] * ncv
    gath = [None] * ncv
    outh = [None] * ncv
    for k in range(ncv + 2):
        if k < ncv:
            p = k % 3
            if k >= 3:
                outh[k - 3].wait()
            b2, r0, ln = chunks[k]
            baseh[k] = pltpu.async_copy(spb.at[pl.ds(r0, ln)],
                                        bufs[p].at[pl.ds(0, ln)], semb[p])
        if 1 <= k < ncv + 1:
            kk = k - 1
            p = kk % 3
            b2, r0, ln = chunks[kk]
            baseh[kk].wait()
            i0t = b2 * JOPAD + r0
            i0m = WIDX + b2 * JOPAD + r0
            gath[kk] = []
            for i0 in (i0t, i0m):
                l1 = min(ln, 128)
                gath[kk].append(pltpu.async_copy(
                    sptm.at[idx_v.at[pl.ds(i0, l1)]],
                    bufs[p].at[pl.ds(0, l1)], semg[p], add=True))
                if ln > 128:
                    gath[kk].append(pltpu.async_copy(
                        sptm.at[idx_v.at[pl.ds(i0 + 128, ln - 128)]],
                        bufs[p].at[pl.ds(128, ln - 128)], semg[p], add=True))
        if 2 <= k < ncv + 2:
            kk = k - 2
            p = kk % 3
            b2, r0, ln = chunks[kk]
            for h in gath[kk]:
                h.wait()
            outh[kk] = pltpu.async_copy(
                bufs[p].at[pl.ds(0, ln)],
                out_hbm.at[b0 + b2, pl.ds(r0, ln)], semo[p])
    for kk in range(ncv - 3, ncv):
        outh[kk].wait()


@jax.jit
def kernel(x, job_table, machine_table, seq_table, W_time, b_time, W_proj, b_proj):
    f32 = jnp.float32
    base_pat, tm_tab = pl.pallas_call(
        _prep_body,
        out_shape=(
            jax.ShapeDtypeStruct((JO, D), f32),
            jax.ShapeDtypeStruct((TMR, D), f32),
        ),
    )(job_table, machine_table, seq_table, W_time,
      b_time.reshape(1, D), W_proj, b_proj.reshape(1, D))

    t_idx = jnp.pad(x[..., 0].reshape(B, JO), ((0, 0), (0, JOPAD - JO)))
    m_idx = jnp.pad(x[..., 1].reshape(B, JO) + 100,
                    ((0, 0), (0, JOPAD - JO)))
    idx = jnp.concatenate(
        [t_idx.reshape(B * JOPAD), m_idx.reshape(B * JOPAD)])

    sc_fn = pl.kernel(
        _sc_body,
        mesh=plsc.VectorSubcoreMesh(core_axis_name="c", subcore_axis_name="s"),
        out_type=jax.ShapeDtypeStruct((B, JO, D), f32),
        scratch_types=(
            [pltpu.VMEM((2 * WIDX,), jnp.int32)]
            + [pltpu.VMEM((CH, D), f32) for _ in range(3)]
            + [pltpu.VMEM_SHARED((TMR, D), f32)]
            + [pltpu.VMEM_SHARED((JO, D), f32)]
            + [pltpu.SemaphoreType.DMA for _ in range(9)]
        ),
    )
    return sc_fn(idx, base_pat, tm_tab)


# speedup vs baseline: 1.3636x; 1.3636x over previous
import jax
import jax.numpy as jnp
from jax import lax
from jax.experimental import pallas as pl
from jax.experimental.pallas import tpu as pltpu
from jax.experimental.pallas import tpu_sc as plsc

B, J, O = 64, 50, 50
D = 128
JO = J * O
JOP = JO + 4
N = B * JO
NC, NS = 2, 16
NW = NC * NS
BPW = B // NW
CH = 64
NCB = 40
JOPAD = 2512
WIDX = BPW * JOPAD
GT = 100 * 100


def _prep_body(job_ref, mach_ref, seq_ref, wt_ref, bt_ref, wp_ref, bp_ref,
               base_ref, tm_ref):
    f32 = jnp.float32
    wp = wp_ref[...]
    w1 = wp[0:128, :]
    w2 = wp[128:256, :]
    w3 = wp[256:384, :]
    w4 = wp[384:512, :]
    const = jnp.dot(bt_ref[...], w4, preferred_element_type=f32) + bp_ref[...]
    a = jnp.dot(job_ref[...][:J], w1, preferred_element_type=f32) + const
    c = jnp.dot(seq_ref[...][:O], w3, preferred_element_type=f32)
    basef = (a[:, None, :] + c[None, :, :]).reshape(JO, D)
    base_ref[...] = jnp.concatenate(
        [basef, jnp.zeros((JOP - JO, D), f32)], axis=0)
    m2 = jnp.dot(mach_ref[...], w2, preferred_element_type=f32)
    v = jnp.dot(wt_ref[...], w4, preferred_element_type=f32)
    t_col = lax.broadcasted_iota(jnp.int32, (100, 1, 1), 0).astype(f32)
    tm_ref[...] = (t_col * v.reshape(1, 1, D) + m2[None, :, :]).reshape(GT, D)


def _sc_body(idx_hbm, base_hbm, tm_hbm, out_hbm,
             idx_v, buf0, buf1, buf2, sptm, spb,
             sb0, sb1, sb2, sg0, sg1, sg2, so0, so1, so2):
    cid = lax.axis_index("c")
    sid = lax.axis_index("s")
    wid = sid * NC + cid
    b0 = wid * BPW

    pltpu.sync_copy(idx_hbm.at[pl.ds(b0 * JOPAD, WIDX)], idx_v)

    @pl.when(sid == 0)
    def _stage():
        pltpu.sync_copy(tm_hbm, sptm)
        pltpu.sync_copy(base_hbm, spb)
    plsc.subcore_barrier()

    bufs = (buf0, buf1, buf2)
    semb = (sb0, sb1, sb2)
    semg = (sg0, sg1, sg2)
    semo = (so0, so1, so2)

    chunks = [(b2, c * CH, CH if c < NCB - 1 else JOP - (NCB - 1) * CH)
              for b2 in range(BPW) for c in range(NCB)]
    ncv = len(chunks)

    baseh = [None] * ncv
    gath = [None] * ncv
    outh = [None] * ncv
    for k in range(ncv + 2):
        if k < ncv:
            p = k % 3
            if k >= 3:
                outh[k - 3].wait()
            b2, r0, ln = chunks[k]
            baseh[k] = pltpu.async_copy(spb.at[pl.ds(r0, ln)],
                                        bufs[p].at[pl.ds(0, ln)], semb[p])
        if 1 <= k < ncv + 1:
            kk = k - 1
            p = kk % 3
            b2, r0, ln = chunks[kk]
            baseh[kk].wait()
            i0 = b2 * JOPAD + r0
            gath[kk] = [pltpu.async_copy(
                sptm.at[idx_v.at[pl.ds(i0, ln)]],
                bufs[p].at[pl.ds(0, ln)], semg[p], add=True)]
        if 2 <= k < ncv + 2:
            kk = k - 2
            p = kk % 3
            b2, r0, ln = chunks[kk]
            for h in gath[kk]:
                h.wait()
            outh[kk] = pltpu.async_copy(
                bufs[p].at[pl.ds(0, ln)],
                out_hbm.at[b0 + b2, pl.ds(r0, ln)], semo[p])
    for kk in range(ncv - 3, ncv):
        outh[kk].wait()


@jax.jit
def kernel(x, job_table, machine_table, seq_table, W_time, b_time, W_proj, b_proj):
    f32 = jnp.float32
    base_pat, tm_tab = pl.pallas_call(
        _prep_body,
        out_shape=(
            jax.ShapeDtypeStruct((JOP, D), f32),
            jax.ShapeDtypeStruct((GT, D), f32),
        ),
    )(job_table, machine_table, seq_table, W_time,
      b_time.reshape(1, D), W_proj, b_proj.reshape(1, D))

    idx = jnp.pad((x[..., 0] * 100 + x[..., 1]).reshape(B, JO),
                  ((0, 0), (0, JOPAD - JO))).reshape(B * JOPAD)

    sc_fn = pl.kernel(
        _sc_body,
        mesh=plsc.VectorSubcoreMesh(core_axis_name="c", subcore_axis_name="s"),
        out_type=jax.ShapeDtypeStruct((B, JOP, D), f32),
        scratch_types=(
            [pltpu.VMEM((WIDX,), jnp.int32)]
            + [pltpu.VMEM((CH, D), f32) for _ in range(3)]
            + [pltpu.VMEM_SHARED((GT, D), f32)]
            + [pltpu.VMEM_SHARED((JOP, D), f32)]
            + [pltpu.SemaphoreType.DMA for _ in range(9)]
        ),
    )
    out = sc_fn(idx, base_pat, tm_tab)
    return out[:, :JO, :]
